# baseline (speedup 1.0000x reference)
import functools

import jax
import jax.numpy as jnp
from jax import lax
from jax.experimental import pallas as pl
from jax.experimental.pallas import tpu as pltpu
from jax.experimental.pallas import tpu_sc as plsc

N = 10000
E = 160000
D = 128
A = 4

_BN = 1000
_BE = 2000

_NC = 2
_NS = 16
_NW = _NC * _NS
_CH = 128
_EPW = E // _NW
_NFULL = _EPW // _CH
_TAIL = _EPW - _NFULL * _CH
_NIDX = _NFULL + 1
_RPT = 632
_NPAD = _RPT * _NS


def _silu(v):
    return v * jax.nn.sigmoid(v)


def _zmul(attr, x):
    return jnp.concatenate([attr[:, j:j + 1] * x for j in range(A)], axis=1)


def _wc(W):
    return W.transpose(2, 1, 0).reshape(W.shape[2] * W.shape[1], W.shape[0])


_BF = jnp.bfloat16


def _emb_body(x_ref, na_ref, w_ref, b_ref, o_ref):
    z = _zmul(na_ref[...].astype(_BF), x_ref[...].astype(_BF))
    o_ref[...] = jnp.dot(z, w_ref[...], preferred_element_type=jnp.float32) + b_ref[...]


def _edge_body(g_ref, ea_ref, w1_ref, b1_ref, w2_ref, b2_ref, o_ref):
    ea = ea_ref[...].astype(_BF)
    xcat = jnp.concatenate([g_ref[0], g_ref[1]], axis=1).astype(_BF)
    m = _silu(jnp.dot(_zmul(ea, xcat), w1_ref[...],
                      preferred_element_type=jnp.float32) + b1_ref[...])
    z2 = _zmul(ea, m.astype(_BF))
    o_ref[...] = _silu(jnp.dot(z2, w2_ref[...],
                               preferred_element_type=jnp.float32) + b2_ref[...])


def _update_body(h_ref, p_ref, na_ref, w1_ref, b1_ref, w2_ref, b2_ref,
                 o_ref):
    na = na_ref[...].astype(_BF)
    h = h_ref[...]
    hcat = jnp.concatenate([h, p_ref[0] + p_ref[1]], axis=1).astype(_BF)
    u = _silu(jnp.dot(_zmul(na, hcat), w1_ref[...],
                      preferred_element_type=jnp.float32) + b1_ref[...])
    o_ref[...] = h + jnp.dot(_zmul(na, u.astype(_BF)), w2_ref[...],
                             preferred_element_type=jnp.float32) + b2_ref[...]


def _head_body(h_ref, na_ref, w1_ref, b1_ref, w2_ref, b2_ref, o_ref):
    na = na_ref[...].astype(_BF)
    u = _silu(jnp.dot(_zmul(na, h_ref[...].astype(_BF)), w1_ref[...],
                      preferred_element_type=jnp.float32) + b1_ref[...])
    o_ref[...] = jnp.dot(_zmul(na, u.astype(_BF)), w2_ref[...],
                         preferred_element_type=jnp.float32) + b2_ref[...]


_SC_MESH = plsc.VectorSubcoreMesh(core_axis_name="c", subcore_axis_name="s")


@functools.partial(
    pl.kernel,
    out_type=jax.ShapeDtypeStruct((_NC, _NPAD, D), jnp.float32),
    mesh=_SC_MESH,
    scratch_types=[
        pltpu.VMEM((_NIDX, _CH), jnp.int32),
        pltpu.VMEM((_CH, D), jnp.float32),
        pltpu.VMEM_SHARED((_NPAD, D), jnp.float32),
    ],
)
def _scatter_sc(m2_hbm, idx_hbm, zeros_hbm, out_hbm, idx_v, buf_v, shared):
    c = lax.axis_index("c")
    s = lax.axis_index("s")
    w = c * _NS + s
    pltpu.sync_copy(idx_hbm.at[w], idx_v)
    pltpu.sync_copy(zeros_hbm.at[pl.ds(s * _RPT, _RPT)],
                    shared.at[pl.ds(s * _RPT, _RPT)])
    plsc.subcore_barrier()
    base = w * _EPW

    def body(t, carry):
        pltpu.sync_copy(m2_hbm.at[pl.ds(base + t * _CH, _CH)], buf_v)
        pltpu.sync_copy(buf_v, shared.at[idx_v.at[t]], add=True)
        return carry

    lax.fori_loop(0, _NFULL, body, 0)
    pltpu.sync_copy(m2_hbm.at[pl.ds(base + _NFULL * _CH, _TAIL)],
                    buf_v.at[pl.ds(0, _TAIL)])
    pltpu.sync_copy(buf_v, shared.at[idx_v.at[_NFULL]], add=True)
    plsc.subcore_barrier()
    pltpu.sync_copy(shared.at[pl.ds(s * _RPT, _RPT)],
                    out_hbm.at[c].at[pl.ds(s * _RPT, _RPT)])


@functools.partial(
    pl.kernel,
    out_type=jax.ShapeDtypeStruct((2, E, D), jnp.float32),
    mesh=_SC_MESH,
    scratch_types=[
        pltpu.VMEM((2 * _NIDX, _CH), jnp.int32),
        pltpu.VMEM((_CH, D), jnp.float32),
        pltpu.VMEM((_CH, D), jnp.float32),
        pltpu.VMEM_SHARED((_NPAD, D), jnp.float32),
        pltpu.SemaphoreType.DMA,
        pltpu.SemaphoreType.DMA,
    ],
)
def _gather_sc(h_hbm, gidx_hbm, out_hbm, idx_v, buf0, buf1, table, sem0, sem1):
    c = lax.axis_index("c")
    s = lax.axis_index("s")
    w = c * _NS + s
    pltpu.sync_copy(gidx_hbm.at[0, w], idx_v.at[pl.ds(0, _NIDX)])
    pltpu.sync_copy(gidx_hbm.at[1, w], idx_v.at[pl.ds(_NIDX, _NIDX)])
    pltpu.sync_copy(h_hbm.at[pl.ds(s * _RPT, _RPT)],
                    table.at[pl.ds(s * _RPT, _RPT)])
    plsc.subcore_barrier()
    wbase = w * _EPW
    dummy = h_hbm.at[pl.ds(0, _CH)]

    for k in range(2):
        rb = k * _NIDX
        pltpu.async_copy(table.at[idx_v.at[rb]], buf0, sem0)

        def body(g, carry, rb=rb, k=k):
            t0 = 2 * g
            pltpu.make_async_copy(dummy, buf0, sem0).wait()
            pltpu.async_copy(table.at[idx_v.at[rb + t0 + 1]], buf1, sem1)
            pltpu.sync_copy(buf0, out_hbm.at[k].at[pl.ds(wbase + t0 * _CH, _CH)])
            pltpu.make_async_copy(dummy, buf1, sem1).wait()
            pltpu.async_copy(table.at[idx_v.at[rb + t0 + 2]], buf0, sem0)
            pltpu.sync_copy(buf1,
                            out_hbm.at[k].at[pl.ds(wbase + (t0 + 1) * _CH, _CH)])
            return carry

        lax.fori_loop(0, (_NFULL - 1) // 2, body, 0)
        pltpu.make_async_copy(dummy, buf0, sem0).wait()
        pltpu.sync_copy(buf0,
                        out_hbm.at[k].at[pl.ds(wbase + (_NFULL - 1) * _CH, _CH)])
        pltpu.sync_copy(table.at[idx_v.at[rb + _NFULL]], buf1)
        pltpu.sync_copy(buf1.at[pl.ds(0, _TAIL)],
                        out_hbm.at[k].at[pl.ds(wbase + _NFULL * _CH, _TAIL)])


def _row_spec(bn, f):
    return pl.BlockSpec((bn, f), lambda i: (i, 0))


def _const_spec(shape):
    return pl.BlockSpec(shape, lambda i: tuple(0 for _ in shape))


def _stack_spec(k, bn, f):
    return pl.BlockSpec((k, bn, f), lambda i: (0, i, 0))


def _call_rows(body, n, bn, ins, row_widths, consts, out_width, n_out=None):
    grid = n // bn
    in_specs = ([_stack_spec(w[0], bn, w[1]) if isinstance(w, tuple)
                 else _row_spec(bn, w) for w in row_widths]
                + [_const_spec(c.shape) for c in consts])
    return pl.pallas_call(
        body,
        grid=(grid,),
        in_specs=in_specs,
        out_specs=_row_spec(bn, out_width),
        out_shape=jax.ShapeDtypeStruct((n_out or n, out_width), jnp.float32),
    )(*ins, *consts)


def kernel(x, edge_index, edge_attr, node_attr, batch, W_emb, b_emb,
           Wm1, bm1, Wm2, bm2, Wu1, bu1, Wu2, bu2, Wp1, bp1, Wp2, bp2):
    L = Wm1.shape[0]
    na = node_attr.at[:, 0].set(1.0)
    src = edge_index[0]
    dst = edge_index[1]

    npad_i = _NIDX * _CH - _EPW
    trash = N + (jnp.arange(_NW * npad_i, dtype=jnp.int32)
                 % (_NPAD - N)).reshape(_NW, npad_i)
    scat_idx = jnp.concatenate(
        [dst.reshape(_NW, _EPW), trash], axis=1).reshape(_NW, _NIDX, _CH)
    gpad = (jnp.arange(_NW * npad_i, dtype=jnp.int32)
            % (_NPAD - N)).reshape(_NW, npad_i)
    gidx = jnp.stack([
        jnp.concatenate([dst.reshape(_NW, _EPW), gpad], axis=1),
        jnp.concatenate([src.reshape(_NW, _EPW), gpad], axis=1),
    ]).reshape(2, _NW, _NIDX, _CH)
    zeros_np = jnp.zeros((_NPAD, D), jnp.float32)

    w_emb = _wc(W_emb).astype(jnp.bfloat16)
    b_embr = b_emb.reshape(1, D)

    h = _call_rows(_emb_body, N, _BN, [x, na], [D, A], [w_emb, b_embr], D,
                   n_out=_NPAD)

    for l in range(L):
        w1 = _wc(Wm1[l]).astype(jnp.bfloat16)
        b1 = bm1[l].reshape(1, D)
        w2 = _wc(Wm2[l]).astype(jnp.bfloat16)
        b2 = bm2[l].reshape(1, D)
        wu1 = _wc(Wu1[l]).astype(jnp.bfloat16)
        bu1r = bu1[l].reshape(1, D)
        wu2 = _wc(Wu2[l]).astype(jnp.bfloat16)
        bu2r = bu2[l].reshape(1, D)

        g = _gather_sc(h, gidx)
        m2 = _call_rows(_edge_body, E, _BE, [g, edge_attr], [(2, D), A],
                        [w1, b1, w2, b2], D)
        parts = _scatter_sc(m2, scat_idx, zeros_np)
        h = _call_rows(_update_body, N, _BN, [h, parts, na],
                       [D, (2, D), A], [wu1, bu1r, wu2, bu2r], D,
                       n_out=_NPAD)

    wp1 = _wc(Wp1).astype(jnp.bfloat16)
    wp2 = _wc(Wp2).astype(jnp.bfloat16)
    h = _call_rows(_head_body, N, _BN, [h, na], [D, A],
                   [wp1, bp1.reshape(1, D), wp2, bp2.reshape(1, D)], D)
    return h


# speedup vs baseline: 1.0647x; 1.0647x over previous
import functools

import jax
import jax.numpy as jnp
from jax import lax
from jax.experimental import pallas as pl
from jax.experimental.pallas import tpu as pltpu
from jax.experimental.pallas import tpu_sc as plsc

N = 10000
E = 160000
D = 128
A = 4

_BN = 1000
_BE = 2000

_NC = 2
_NS = 16
_NW = _NC * _NS
_CH = 128
_EPW = E // _NW
_NFULL = _EPW // _CH
_TAIL = _EPW - _NFULL * _CH
_NIDX = _NFULL + 1
_RPT = 632
_NPAD = _RPT * _NS


def _silu(v):
    return v * jax.nn.sigmoid(v)


def _zmul(attr, x):
    return jnp.concatenate([attr[:, j:j + 1] * x for j in range(A)], axis=1)


def _wc(W):
    return W.transpose(2, 1, 0).reshape(W.shape[2] * W.shape[1], W.shape[0])


def _emb_body(x_ref, na_ref, w_ref, b_ref, o_ref):
    z = _zmul(na_ref[...], x_ref[...])
    o_ref[...] = jnp.dot(z, w_ref[...], preferred_element_type=jnp.float32) + b_ref[...]


def _edge_body(g_ref, ea_ref, w1_ref, b1_ref, w2_ref, b2_ref, o_ref):
    ea = ea_ref[...]
    xcat = jnp.concatenate([g_ref[0], g_ref[1]], axis=1)
    z1 = _zmul(ea, xcat).astype(jnp.bfloat16)
    m = _silu(jnp.dot(z1, w1_ref[...],
                      preferred_element_type=jnp.float32) + b1_ref[...])
    z2 = _zmul(ea, m).astype(jnp.bfloat16)
    o_ref[...] = _silu(jnp.dot(z2, w2_ref[...],
                               preferred_element_type=jnp.float32) + b2_ref[...])


def _update_body(h_ref, p_ref, na_ref, w1_ref, b1_ref, w2_ref, b2_ref,
                 o_ref):
    na = na_ref[...]
    h = h_ref[...]
    hcat = jnp.concatenate([h, p_ref[0] + p_ref[1]], axis=1)
    u = _silu(jnp.dot(_zmul(na, hcat), w1_ref[...],
                      preferred_element_type=jnp.float32) + b1_ref[...])
    o_ref[...] = h + jnp.dot(_zmul(na, u), w2_ref[...],
                             preferred_element_type=jnp.float32) + b2_ref[...]


def _head_body(h_ref, na_ref, w1_ref, b1_ref, w2_ref, b2_ref, o_ref):
    na = na_ref[...]
    u = _silu(jnp.dot(_zmul(na, h_ref[...]), w1_ref[...],
                      preferred_element_type=jnp.float32) + b1_ref[...])
    o_ref[...] = jnp.dot(_zmul(na, u), w2_ref[...],
                         preferred_element_type=jnp.float32) + b2_ref[...]


_SC_MESH = plsc.VectorSubcoreMesh(core_axis_name="c", subcore_axis_name="s")


@functools.partial(
    pl.kernel,
    out_type=jax.ShapeDtypeStruct((_NC, _NPAD, D), jnp.float32),
    mesh=_SC_MESH,
    scratch_types=[
        pltpu.VMEM((_NIDX, _CH), jnp.int32),
        pltpu.VMEM((_CH, D), jnp.float32),
        pltpu.VMEM((_CH, D), jnp.float32),
        pltpu.VMEM_SHARED((_NPAD, D), jnp.float32),
        pltpu.SemaphoreType.DMA,
        pltpu.SemaphoreType.DMA,
    ],
)
def _scatter_sc(m2_hbm, idx_hbm, zeros_hbm, out_hbm, idx_v, buf0, buf1,
                shared, sem0, sem1):
    c = lax.axis_index("c")
    s = lax.axis_index("s")
    w = c * _NS + s
    pltpu.sync_copy(idx_hbm.at[w], idx_v)
    pltpu.sync_copy(zeros_hbm.at[pl.ds(s * _RPT, _RPT)],
                    shared.at[pl.ds(s * _RPT, _RPT)])
    plsc.subcore_barrier()
    base = w * _EPW
    dummy = m2_hbm.at[pl.ds(0, _CH)]
    pltpu.async_copy(m2_hbm.at[pl.ds(base, _CH)], buf0, sem0)

    def body(g, carry):
        t0 = 2 * g
        pltpu.make_async_copy(dummy, buf0, sem0).wait()
        pltpu.async_copy(m2_hbm.at[pl.ds(base + (t0 + 1) * _CH, _CH)],
                         buf1, sem1)
        pltpu.sync_copy(buf0, shared.at[idx_v.at[t0]], add=True)
        pltpu.make_async_copy(dummy, buf1, sem1).wait()
        pltpu.async_copy(m2_hbm.at[pl.ds(base + (t0 + 2) * _CH, _CH)],
                         buf0, sem0)
        pltpu.sync_copy(buf1, shared.at[idx_v.at[t0 + 1]], add=True)
        return carry

    lax.fori_loop(0, (_NFULL - 1) // 2, body, 0)
    pltpu.make_async_copy(dummy, buf0, sem0).wait()
    pltpu.sync_copy(buf0, shared.at[idx_v.at[_NFULL - 1]], add=True)
    pltpu.sync_copy(m2_hbm.at[pl.ds(base + _NFULL * _CH, _TAIL)],
                    buf1.at[pl.ds(0, _TAIL)])
    pltpu.sync_copy(buf1, shared.at[idx_v.at[_NFULL]], add=True)
    plsc.subcore_barrier()
    pltpu.sync_copy(shared.at[pl.ds(s * _RPT, _RPT)],
                    out_hbm.at[c].at[pl.ds(s * _RPT, _RPT)])


@functools.partial(
    pl.kernel,
    out_type=jax.ShapeDtypeStruct((2, E, D), jnp.float32),
    mesh=_SC_MESH,
    scratch_types=[
        pltpu.VMEM((2 * _NIDX, _CH), jnp.int32),
        pltpu.VMEM((_CH, D), jnp.float32),
        pltpu.VMEM((_CH, D), jnp.float32),
        pltpu.VMEM_SHARED((_NPAD, D), jnp.float32),
        pltpu.SemaphoreType.DMA,
        pltpu.SemaphoreType.DMA,
    ],
)
def _gather_sc(h_hbm, gidx_hbm, out_hbm, idx_v, buf0, buf1, table, sem0, sem1):
    c = lax.axis_index("c")
    s = lax.axis_index("s")
    w = c * _NS + s
    pltpu.sync_copy(gidx_hbm.at[0, w], idx_v.at[pl.ds(0, _NIDX)])
    pltpu.sync_copy(gidx_hbm.at[1, w], idx_v.at[pl.ds(_NIDX, _NIDX)])
    pltpu.sync_copy(h_hbm.at[pl.ds(s * _RPT, _RPT)],
                    table.at[pl.ds(s * _RPT, _RPT)])
    plsc.subcore_barrier()
    wbase = w * _EPW
    dummy = h_hbm.at[pl.ds(0, _CH)]

    for k in range(2):
        rb = k * _NIDX
        pltpu.async_copy(table.at[idx_v.at[rb]], buf0, sem0)

        def body(g, carry, rb=rb, k=k):
            t0 = 2 * g
            pltpu.make_async_copy(dummy, buf0, sem0).wait()
            pltpu.async_copy(table.at[idx_v.at[rb + t0 + 1]], buf1, sem1)
            pltpu.sync_copy(buf0, out_hbm.at[k].at[pl.ds(wbase + t0 * _CH, _CH)])
            pltpu.make_async_copy(dummy, buf1, sem1).wait()
            pltpu.async_copy(table.at[idx_v.at[rb + t0 + 2]], buf0, sem0)
            pltpu.sync_copy(buf1,
                            out_hbm.at[k].at[pl.ds(wbase + (t0 + 1) * _CH, _CH)])
            return carry

        lax.fori_loop(0, (_NFULL - 1) // 2, body, 0)
        pltpu.make_async_copy(dummy, buf0, sem0).wait()
        pltpu.sync_copy(buf0,
                        out_hbm.at[k].at[pl.ds(wbase + (_NFULL - 1) * _CH, _CH)])
        pltpu.sync_copy(table.at[idx_v.at[rb + _NFULL]], buf1)
        pltpu.sync_copy(buf1.at[pl.ds(0, _TAIL)],
                        out_hbm.at[k].at[pl.ds(wbase + _NFULL * _CH, _TAIL)])


def _row_spec(bn, f):
    return pl.BlockSpec((bn, f), lambda i: (i, 0))


def _const_spec(shape):
    return pl.BlockSpec(shape, lambda i: tuple(0 for _ in shape))


def _stack_spec(k, bn, f):
    return pl.BlockSpec((k, bn, f), lambda i: (0, i, 0))


def _call_rows(body, n, bn, ins, row_widths, consts, out_width, n_out=None):
    grid = n // bn
    in_specs = ([_stack_spec(w[0], bn, w[1]) if isinstance(w, tuple)
                 else _row_spec(bn, w) for w in row_widths]
                + [_const_spec(c.shape) for c in consts])
    return pl.pallas_call(
        body,
        grid=(grid,),
        in_specs=in_specs,
        out_specs=_row_spec(bn, out_width),
        out_shape=jax.ShapeDtypeStruct((n_out or n, out_width), jnp.float32),
    )(*ins, *consts)


def kernel(x, edge_index, edge_attr, node_attr, batch, W_emb, b_emb,
           Wm1, bm1, Wm2, bm2, Wu1, bu1, Wu2, bu2, Wp1, bp1, Wp2, bp2):
    L = Wm1.shape[0]
    na = node_attr.at[:, 0].set(1.0)
    src = edge_index[0]
    dst = edge_index[1]

    npad_i = _NIDX * _CH - _EPW
    trash = N + (jnp.arange(_NW * npad_i, dtype=jnp.int32)
                 % (_NPAD - N)).reshape(_NW, npad_i)
    scat_idx = jnp.concatenate(
        [dst.reshape(_NW, _EPW), trash], axis=1).reshape(_NW, _NIDX, _CH)
    gpad = (jnp.arange(_NW * npad_i, dtype=jnp.int32)
            % (_NPAD - N)).reshape(_NW, npad_i)
    gidx = jnp.stack([
        jnp.concatenate([dst.reshape(_NW, _EPW), gpad], axis=1),
        jnp.concatenate([src.reshape(_NW, _EPW), gpad], axis=1),
    ]).reshape(2, _NW, _NIDX, _CH)
    zeros_np = jnp.zeros((_NPAD, D), jnp.float32)

    w_emb = _wc(W_emb)
    b_embr = b_emb.reshape(1, D)

    h = _call_rows(_emb_body, N, _BN, [x, na], [D, A], [w_emb, b_embr], D,
                   n_out=_NPAD)

    for l in range(L):
        w1 = _wc(Wm1[l]).astype(jnp.bfloat16)
        b1 = bm1[l].reshape(1, D)
        w2 = _wc(Wm2[l]).astype(jnp.bfloat16)
        b2 = bm2[l].reshape(1, D)
        wu1 = _wc(Wu1[l])
        bu1r = bu1[l].reshape(1, D)
        wu2 = _wc(Wu2[l])
        bu2r = bu2[l].reshape(1, D)

        g = _gather_sc(h, gidx)
        m2 = _call_rows(_edge_body, E, _BE, [g, edge_attr], [(2, D), A],
                        [w1, b1, w2, b2], D)
        parts = _scatter_sc(m2, scat_idx, zeros_np)
        h = _call_rows(_update_body, N, _BN, [h, parts, na],
                       [D, (2, D), A], [wu1, bu1r, wu2, bu2r], D,
                       n_out=_NPAD)

    wp1 = _wc(Wp1)
    wp2 = _wc(Wp2)
    h = _call_rows(_head_body, N, _BN, [h, na], [D, A],
                   [wp1, bp1.reshape(1, D), wp2, bp2.reshape(1, D)], D)
    return h


# speedup vs baseline: 1.0740x; 1.0088x over previous
import functools

import jax
import jax.numpy as jnp
from jax import lax
from jax.experimental import pallas as pl
from jax.experimental.pallas import tpu as pltpu
from jax.experimental.pallas import tpu_sc as plsc

N = 10000
E = 160000
D = 128
A = 4

_BN = 1000
_BE = 2000

_NC = 2
_NS = 16
_NW = _NC * _NS
_CH = 128
_EPW = E // _NW
_NFULL = _EPW // _CH
_TAIL = _EPW - _NFULL * _CH
_NIDX = _NFULL + 1
_RPT = 632
_NPAD = _RPT * _NS


def _silu(v):
    return v * jax.nn.sigmoid(v)


def _zmul(attr, x):
    return jnp.concatenate([attr[:, j:j + 1] * x for j in range(A)], axis=1)


def _wc(W):
    return W.transpose(2, 1, 0).reshape(W.shape[2] * W.shape[1], W.shape[0])


def _emb_body(x_ref, na_ref, w_ref, b_ref, o_ref):
    z = _zmul(na_ref[...], x_ref[...])
    o_ref[...] = jnp.dot(z, w_ref[...], preferred_element_type=jnp.float32) + b_ref[...]


def _edge_body(g_ref, ea_ref, w1_ref, b1_ref, w2_ref, b2_ref, o_ref):
    ea = ea_ref[...]
    xcat = jnp.concatenate([g_ref[0], g_ref[1]], axis=1)
    z1 = _zmul(ea, xcat).astype(jnp.bfloat16)
    m = _silu(jnp.dot(z1, w1_ref[...],
                      preferred_element_type=jnp.float32) + b1_ref[...])
    z2 = _zmul(ea, m).astype(jnp.bfloat16)
    o_ref[...] = _silu(jnp.dot(z2, w2_ref[...],
                               preferred_element_type=jnp.float32) + b2_ref[...])


def _update_body(h_ref, p_ref, na_ref, w1_ref, b1_ref, w2_ref, b2_ref,
                 o_ref):
    na = na_ref[...]
    h = h_ref[...]
    hcat = jnp.concatenate([h, p_ref[0] + p_ref[1]], axis=1)
    u = _silu(jnp.dot(_zmul(na, hcat), w1_ref[...],
                      preferred_element_type=jnp.float32) + b1_ref[...])
    o_ref[...] = h + jnp.dot(_zmul(na, u), w2_ref[...],
                             preferred_element_type=jnp.float32) + b2_ref[...]


def _update_head_body(h_ref, p_ref, na_ref, wu1_ref, bu1_ref, wu2_ref,
                      bu2_ref, wp1_ref, bp1_ref, wp2_ref, bp2_ref, o_ref):
    na = na_ref[...]
    h = h_ref[...]
    hcat = jnp.concatenate([h, p_ref[0] + p_ref[1]], axis=1)
    u = _silu(jnp.dot(_zmul(na, hcat), wu1_ref[...],
                      preferred_element_type=jnp.float32) + bu1_ref[...])
    hn = h + jnp.dot(_zmul(na, u), wu2_ref[...],
                     preferred_element_type=jnp.float32) + bu2_ref[...]
    v = _silu(jnp.dot(_zmul(na, hn), wp1_ref[...],
                      preferred_element_type=jnp.float32) + bp1_ref[...])
    o_ref[...] = jnp.dot(_zmul(na, v), wp2_ref[...],
                         preferred_element_type=jnp.float32) + bp2_ref[...]


_SC_MESH = plsc.VectorSubcoreMesh(core_axis_name="c", subcore_axis_name="s")


@functools.partial(
    pl.kernel,
    out_type=jax.ShapeDtypeStruct((_NC, _NPAD, D), jnp.float32),
    mesh=_SC_MESH,
    scratch_types=[
        pltpu.VMEM((_NIDX, _CH), jnp.int32),
        pltpu.VMEM((_CH, D), jnp.float32),
        pltpu.VMEM((_CH, D), jnp.float32),
        pltpu.VMEM_SHARED((_NPAD, D), jnp.float32),
        pltpu.SemaphoreType.DMA,
        pltpu.SemaphoreType.DMA,
    ],
)
def _scatter_sc(m2_hbm, idx_hbm, zeros_hbm, out_hbm, idx_v, buf0, buf1,
                shared, sem0, sem1):
    c = lax.axis_index("c")
    s = lax.axis_index("s")
    w = c * _NS + s
    pltpu.sync_copy(idx_hbm.at[w], idx_v)
    pltpu.sync_copy(zeros_hbm.at[pl.ds(s * _RPT, _RPT)],
                    shared.at[pl.ds(s * _RPT, _RPT)])
    plsc.subcore_barrier()
    base = w * _EPW
    dummy = m2_hbm.at[pl.ds(0, _CH)]
    pltpu.async_copy(m2_hbm.at[pl.ds(base, _CH)], buf0, sem0)

    def body(g, carry):
        t0 = 2 * g
        pltpu.make_async_copy(dummy, buf0, sem0).wait()
        pltpu.async_copy(m2_hbm.at[pl.ds(base + (t0 + 1) * _CH, _CH)],
                         buf1, sem1)
        pltpu.sync_copy(buf0, shared.at[idx_v.at[t0]], add=True)
        pltpu.make_async_copy(dummy, buf1, sem1).wait()
        pltpu.async_copy(m2_hbm.at[pl.ds(base + (t0 + 2) * _CH, _CH)],
                         buf0, sem0)
        pltpu.sync_copy(buf1, shared.at[idx_v.at[t0 + 1]], add=True)
        return carry

    lax.fori_loop(0, (_NFULL - 1) // 2, body, 0)
    pltpu.make_async_copy(dummy, buf0, sem0).wait()
    pltpu.sync_copy(buf0, shared.at[idx_v.at[_NFULL - 1]], add=True)
    pltpu.sync_copy(m2_hbm.at[pl.ds(base + _NFULL * _CH, _TAIL)],
                    buf1.at[pl.ds(0, _TAIL)])
    pltpu.sync_copy(buf1, shared.at[idx_v.at[_NFULL]], add=True)
    plsc.subcore_barrier()
    pltpu.sync_copy(shared.at[pl.ds(s * _RPT, _RPT)],
                    out_hbm.at[c].at[pl.ds(s * _RPT, _RPT)])


@functools.partial(
    pl.kernel,
    out_type=jax.ShapeDtypeStruct((2, E, D), jnp.float32),
    mesh=_SC_MESH,
    scratch_types=[
        pltpu.VMEM((2 * _NIDX, _CH), jnp.int32),
        pltpu.VMEM((_CH, D), jnp.float32),
        pltpu.VMEM((_CH, D), jnp.float32),
        pltpu.VMEM_SHARED((_NPAD, D), jnp.float32),
        pltpu.SemaphoreType.DMA,
        pltpu.SemaphoreType.DMA,
    ],
)
def _gather_sc(h_hbm, gidx_hbm, out_hbm, idx_v, buf0, buf1, table, sem0, sem1):
    c = lax.axis_index("c")
    s = lax.axis_index("s")
    w = c * _NS + s
    pltpu.sync_copy(gidx_hbm.at[0, w], idx_v.at[pl.ds(0, _NIDX)])
    pltpu.sync_copy(gidx_hbm.at[1, w], idx_v.at[pl.ds(_NIDX, _NIDX)])
    pltpu.sync_copy(h_hbm.at[pl.ds(s * _RPT, _RPT)],
                    table.at[pl.ds(s * _RPT, _RPT)])
    plsc.subcore_barrier()
    wbase = w * _EPW
    dummy = h_hbm.at[pl.ds(0, _CH)]

    for k in range(2):
        rb = k * _NIDX
        pltpu.async_copy(table.at[idx_v.at[rb]], buf0, sem0)

        def body(g, carry, rb=rb, k=k):
            t0 = 2 * g
            pltpu.make_async_copy(dummy, buf0, sem0).wait()
            pltpu.async_copy(table.at[idx_v.at[rb + t0 + 1]], buf1, sem1)
            pltpu.sync_copy(buf0, out_hbm.at[k].at[pl.ds(wbase + t0 * _CH, _CH)])
            pltpu.make_async_copy(dummy, buf1, sem1).wait()
            pltpu.async_copy(table.at[idx_v.at[rb + t0 + 2]], buf0, sem0)
            pltpu.sync_copy(buf1,
                            out_hbm.at[k].at[pl.ds(wbase + (t0 + 1) * _CH, _CH)])
            return carry

        lax.fori_loop(0, (_NFULL - 1) // 2, body, 0)
        pltpu.make_async_copy(dummy, buf0, sem0).wait()
        pltpu.sync_copy(buf0,
                        out_hbm.at[k].at[pl.ds(wbase + (_NFULL - 1) * _CH, _CH)])
        pltpu.sync_copy(table.at[idx_v.at[rb + _NFULL]], buf1)
        pltpu.sync_copy(buf1.at[pl.ds(0, _TAIL)],
                        out_hbm.at[k].at[pl.ds(wbase + _NFULL * _CH, _TAIL)])


def _row_spec(bn, f):
    return pl.BlockSpec((bn, f), lambda i: (i, 0))


def _const_spec(shape):
    return pl.BlockSpec(shape, lambda i: tuple(0 for _ in shape))


def _stack_spec(k, bn, f):
    return pl.BlockSpec((k, bn, f), lambda i: (0, i, 0))


def _call_rows(body, n, bn, ins, row_widths, consts, out_width, n_out=None):
    grid = n // bn
    in_specs = ([_stack_spec(w[0], bn, w[1]) if isinstance(w, tuple)
                 else _row_spec(bn, w) for w in row_widths]
                + [_const_spec(c.shape) for c in consts])
    return pl.pallas_call(
        body,
        grid=(grid,),
        in_specs=in_specs,
        out_specs=_row_spec(bn, out_width),
        out_shape=jax.ShapeDtypeStruct((n_out or n, out_width), jnp.float32),
    )(*ins, *consts)


def kernel(x, edge_index, edge_attr, node_attr, batch, W_emb, b_emb,
           Wm1, bm1, Wm2, bm2, Wu1, bu1, Wu2, bu2, Wp1, bp1, Wp2, bp2):
    L = Wm1.shape[0]
    na = node_attr.at[:, 0].set(1.0)
    src = edge_index[0]
    dst = edge_index[1]

    npad_i = _NIDX * _CH - _EPW
    trash = N + (jnp.arange(_NW * npad_i, dtype=jnp.int32)
                 % (_NPAD - N)).reshape(_NW, npad_i)
    scat_idx = jnp.concatenate(
        [dst.reshape(_NW, _EPW), trash], axis=1).reshape(_NW, _NIDX, _CH)
    gpad = (jnp.arange(_NW * npad_i, dtype=jnp.int32)
            % (_NPAD - N)).reshape(_NW, npad_i)
    gidx = jnp.stack([
        jnp.concatenate([dst.reshape(_NW, _EPW), gpad], axis=1),
        jnp.concatenate([src.reshape(_NW, _EPW), gpad], axis=1),
    ]).reshape(2, _NW, _NIDX, _CH)
    zeros_np = jnp.zeros((_NPAD, D), jnp.float32)

    w_emb = _wc(W_emb)
    b_embr = b_emb.reshape(1, D)

    h = _call_rows(_emb_body, N, _BN, [x, na], [D, A], [w_emb, b_embr], D,
                   n_out=_NPAD)

    for l in range(L):
        w1 = _wc(Wm1[l]).astype(jnp.bfloat16)
        b1 = bm1[l].reshape(1, D)
        w2 = _wc(Wm2[l]).astype(jnp.bfloat16)
        b2 = bm2[l].reshape(1, D)
        wu1 = _wc(Wu1[l])
        bu1r = bu1[l].reshape(1, D)
        wu2 = _wc(Wu2[l])
        bu2r = bu2[l].reshape(1, D)

        g = _gather_sc(h, gidx)
        m2 = _call_rows(_edge_body, E, _BE, [g, edge_attr], [(2, D), A],
                        [w1, b1, w2, b2], D)
        parts = _scatter_sc(m2, scat_idx, zeros_np)
        if l < L - 1:
            h = _call_rows(_update_body, N, _BN, [h, parts, na],
                           [D, (2, D), A], [wu1, bu1r, wu2, bu2r], D,
                           n_out=_NPAD)
        else:
            wp1 = _wc(Wp1)
            wp2 = _wc(Wp2)
            h = _call_rows(_update_head_body, N, _BN, [h, parts, na],
                           [D, (2, D), A],
                           [wu1, bu1r, wu2, bu2r,
                            wp1, bp1.reshape(1, D), wp2, bp2.reshape(1, D)],
                           D)
    return h


# speedup vs baseline: 1.1141x; 1.0373x over previous
import functools

import jax
import jax.numpy as jnp
from jax import lax
from jax.experimental import pallas as pl
from jax.experimental.pallas import tpu as pltpu
from jax.experimental.pallas import tpu_sc as plsc

N = 10000
E = 160000
D = 128
A = 4

_BN = 1000
_BE = 4000

_NC = 2
_NS = 16
_NW = _NC * _NS
_CH = 128
_EPW = E // _NW
_NFULL = _EPW // _CH
_TAIL = _EPW - _NFULL * _CH
_NIDX = _NFULL + 1
_RPT = 632
_NPAD = _RPT * _NS


def _silu(v):
    return v * jax.nn.sigmoid(v)


def _zmul(attr, x):
    return jnp.concatenate([attr[:, j:j + 1] * x for j in range(A)], axis=1)


def _wc(W):
    return W.transpose(2, 1, 0).reshape(W.shape[2] * W.shape[1], W.shape[0])


def _emb_body(x_ref, na_ref, w_ref, b_ref, o_ref):
    z = _zmul(na_ref[...], x_ref[...])
    o_ref[...] = jnp.dot(z, w_ref[...], preferred_element_type=jnp.float32) + b_ref[...]


def _edge_body(g_ref, ea_ref, w1_ref, b1_ref, w2_ref, b2_ref, o_ref):
    ea = ea_ref[...]
    xcat = jnp.concatenate([g_ref[0], g_ref[1]], axis=1)
    z1 = _zmul(ea, xcat).astype(jnp.bfloat16)
    m = _silu(jnp.dot(z1, w1_ref[...],
                      preferred_element_type=jnp.float32) + b1_ref[...])
    z2 = _zmul(ea, m).astype(jnp.bfloat16)
    o_ref[...] = _silu(jnp.dot(z2, w2_ref[...],
                               preferred_element_type=jnp.float32) + b2_ref[...])


def _update_body(h_ref, p_ref, na_ref, w1_ref, b1_ref, w2_ref, b2_ref,
                 o_ref):
    na = na_ref[...]
    h = h_ref[...]
    hcat = jnp.concatenate([h, p_ref[0] + p_ref[1]], axis=1)
    u = _silu(jnp.dot(_zmul(na, hcat), w1_ref[...],
                      preferred_element_type=jnp.float32) + b1_ref[...])
    o_ref[...] = h + jnp.dot(_zmul(na, u), w2_ref[...],
                             preferred_element_type=jnp.float32) + b2_ref[...]


def _update_head_body(h_ref, p_ref, na_ref, wu1_ref, bu1_ref, wu2_ref,
                      bu2_ref, wp1_ref, bp1_ref, wp2_ref, bp2_ref, o_ref):
    na = na_ref[...]
    h = h_ref[...]
    hcat = jnp.concatenate([h, p_ref[0] + p_ref[1]], axis=1)
    u = _silu(jnp.dot(_zmul(na, hcat), wu1_ref[...],
                      preferred_element_type=jnp.float32) + bu1_ref[...])
    hn = h + jnp.dot(_zmul(na, u), wu2_ref[...],
                     preferred_element_type=jnp.float32) + bu2_ref[...]
    v = _silu(jnp.dot(_zmul(na, hn), wp1_ref[...],
                      preferred_element_type=jnp.float32) + bp1_ref[...])
    o_ref[...] = jnp.dot(_zmul(na, v), wp2_ref[...],
                         preferred_element_type=jnp.float32) + bp2_ref[...]


_SC_MESH = plsc.VectorSubcoreMesh(core_axis_name="c", subcore_axis_name="s")


@functools.partial(
    pl.kernel,
    out_type=jax.ShapeDtypeStruct((_NC, _NPAD, D), jnp.float32),
    mesh=_SC_MESH,
    scratch_types=[
        pltpu.VMEM((_NIDX, _CH), jnp.int32),
        pltpu.VMEM((_CH, D), jnp.float32),
        pltpu.VMEM((_CH, D), jnp.float32),
        pltpu.VMEM_SHARED((_NPAD, D), jnp.float32),
        pltpu.SemaphoreType.DMA,
        pltpu.SemaphoreType.DMA,
    ],
)
def _scatter_sc(m2_hbm, idx_hbm, zeros_hbm, out_hbm, idx_v, buf0, buf1,
                shared, sem0, sem1):
    c = lax.axis_index("c")
    s = lax.axis_index("s")
    w = c * _NS + s
    pltpu.sync_copy(idx_hbm.at[w], idx_v)
    pltpu.sync_copy(zeros_hbm.at[pl.ds(s * _RPT, _RPT)],
                    shared.at[pl.ds(s * _RPT, _RPT)])
    plsc.subcore_barrier()
    base = w * _EPW
    dummy = m2_hbm.at[pl.ds(0, _CH)]
    pltpu.async_copy(m2_hbm.at[pl.ds(base, _CH)], buf0, sem0)

    def body(g, carry):
        t0 = 2 * g
        pltpu.make_async_copy(dummy, buf0, sem0).wait()
        pltpu.async_copy(m2_hbm.at[pl.ds(base + (t0 + 1) * _CH, _CH)],
                         buf1, sem1)
        pltpu.sync_copy(buf0, shared.at[idx_v.at[t0]], add=True)
        pltpu.make_async_copy(dummy, buf1, sem1).wait()
        pltpu.async_copy(m2_hbm.at[pl.ds(base + (t0 + 2) * _CH, _CH)],
                         buf0, sem0)
        pltpu.sync_copy(buf1, shared.at[idx_v.at[t0 + 1]], add=True)
        return carry

    lax.fori_loop(0, (_NFULL - 1) // 2, body, 0)
    pltpu.make_async_copy(dummy, buf0, sem0).wait()
    pltpu.sync_copy(buf0, shared.at[idx_v.at[_NFULL - 1]], add=True)
    pltpu.sync_copy(m2_hbm.at[pl.ds(base + _NFULL * _CH, _TAIL)],
                    buf1.at[pl.ds(0, _TAIL)])
    pltpu.sync_copy(buf1, shared.at[idx_v.at[_NFULL]], add=True)
    plsc.subcore_barrier()
    pltpu.sync_copy(shared.at[pl.ds(s * _RPT, _RPT)],
                    out_hbm.at[c].at[pl.ds(s * _RPT, _RPT)])


@functools.partial(
    pl.kernel,
    out_type=jax.ShapeDtypeStruct((2, E, D), jnp.float32),
    mesh=_SC_MESH,
    scratch_types=[
        pltpu.VMEM((2 * _NIDX, _CH), jnp.int32),
        pltpu.VMEM((_CH, D), jnp.float32),
        pltpu.VMEM((_CH, D), jnp.float32),
        pltpu.VMEM_SHARED((_NPAD, D), jnp.float32),
        pltpu.SemaphoreType.DMA,
        pltpu.SemaphoreType.DMA,
    ],
)
def _gather_sc(h_hbm, gidx_hbm, out_hbm, idx_v, buf0, buf1, table, sem0, sem1):
    c = lax.axis_index("c")
    s = lax.axis_index("s")
    w = c * _NS + s
    pltpu.sync_copy(gidx_hbm.at[0, w], idx_v.at[pl.ds(0, _NIDX)])
    pltpu.sync_copy(gidx_hbm.at[1, w], idx_v.at[pl.ds(_NIDX, _NIDX)])
    pltpu.sync_copy(h_hbm.at[pl.ds(s * _RPT, _RPT)],
                    table.at[pl.ds(s * _RPT, _RPT)])
    plsc.subcore_barrier()
    wbase = w * _EPW
    dummy = h_hbm.at[pl.ds(0, _CH)]

    for k in range(2):
        rb = k * _NIDX
        pltpu.async_copy(table.at[idx_v.at[rb]], buf0, sem0)

        def body(g, carry, rb=rb, k=k):
            t0 = 2 * g
            pltpu.make_async_copy(dummy, buf0, sem0).wait()
            pltpu.async_copy(table.at[idx_v.at[rb + t0 + 1]], buf1, sem1)
            pltpu.sync_copy(buf0, out_hbm.at[k].at[pl.ds(wbase + t0 * _CH, _CH)])
            pltpu.make_async_copy(dummy, buf1, sem1).wait()
            pltpu.async_copy(table.at[idx_v.at[rb + t0 + 2]], buf0, sem0)
            pltpu.sync_copy(buf1,
                            out_hbm.at[k].at[pl.ds(wbase + (t0 + 1) * _CH, _CH)])
            return carry

        lax.fori_loop(0, (_NFULL - 1) // 2, body, 0)
        pltpu.make_async_copy(dummy, buf0, sem0).wait()
        pltpu.sync_copy(buf0,
                        out_hbm.at[k].at[pl.ds(wbase + (_NFULL - 1) * _CH, _CH)])
        pltpu.sync_copy(table.at[idx_v.at[rb + _NFULL]], buf1)
        pltpu.sync_copy(buf1.at[pl.ds(0, _TAIL)],
                        out_hbm.at[k].at[pl.ds(wbase + _NFULL * _CH, _TAIL)])


def _row_spec(bn, f):
    return pl.BlockSpec((bn, f), lambda i: (i, 0))


def _const_spec(shape):
    return pl.BlockSpec(shape, lambda i: tuple(0 for _ in shape))


def _stack_spec(k, bn, f):
    return pl.BlockSpec((k, bn, f), lambda i: (0, i, 0))


def _call_rows(body, n, bn, ins, row_widths, consts, out_width, n_out=None):
    grid = n // bn
    in_specs = ([_stack_spec(w[0], bn, w[1]) if isinstance(w, tuple)
                 else _row_spec(bn, w) for w in row_widths]
                + [_const_spec(c.shape) for c in consts])
    return pl.pallas_call(
        body,
        grid=(grid,),
        in_specs=in_specs,
        out_specs=_row_spec(bn, out_width),
        out_shape=jax.ShapeDtypeStruct((n_out or n, out_width), jnp.float32),
    )(*ins, *consts)


def kernel(x, edge_index, edge_attr, node_attr, batch, W_emb, b_emb,
           Wm1, bm1, Wm2, bm2, Wu1, bu1, Wu2, bu2, Wp1, bp1, Wp2, bp2):
    L = Wm1.shape[0]
    na = node_attr.at[:, 0].set(1.0)
    src = edge_index[0]
    dst = edge_index[1]

    npad_i = _NIDX * _CH - _EPW
    trash = N + (jnp.arange(_NW * npad_i, dtype=jnp.int32)
                 % (_NPAD - N)).reshape(_NW, npad_i)
    scat_idx = jnp.concatenate(
        [dst.reshape(_NW, _EPW), trash], axis=1).reshape(_NW, _NIDX, _CH)
    gpad = (jnp.arange(_NW * npad_i, dtype=jnp.int32)
            % (_NPAD - N)).reshape(_NW, npad_i)
    gidx = jnp.stack([
        jnp.concatenate([dst.reshape(_NW, _EPW), gpad], axis=1),
        jnp.concatenate([src.reshape(_NW, _EPW), gpad], axis=1),
    ]).reshape(2, _NW, _NIDX, _CH)
    zeros_np = jnp.zeros((_NPAD, D), jnp.float32)

    w_emb = _wc(W_emb)
    b_embr = b_emb.reshape(1, D)

    h = _call_rows(_emb_body, N, _BN, [x, na], [D, A], [w_emb, b_embr], D,
                   n_out=_NPAD)

    for l in range(L):
        w1 = _wc(Wm1[l]).astype(jnp.bfloat16)
        b1 = bm1[l].reshape(1, D)
        w2 = _wc(Wm2[l]).astype(jnp.bfloat16)
        b2 = bm2[l].reshape(1, D)
        wu1 = _wc(Wu1[l])
        bu1r = bu1[l].reshape(1, D)
        wu2 = _wc(Wu2[l])
        bu2r = bu2[l].reshape(1, D)

        g = _gather_sc(h, gidx)
        m2 = _call_rows(_edge_body, E, _BE, [g, edge_attr], [(2, D), A],
                        [w1, b1, w2, b2], D)
        parts = _scatter_sc(m2, scat_idx, zeros_np)
        if l < L - 1:
            h = _call_rows(_update_body, N, _BN, [h, parts, na],
                           [D, (2, D), A], [wu1, bu1r, wu2, bu2r], D,
                           n_out=_NPAD)
        else:
            wp1 = _wc(Wp1)
            wp2 = _wc(Wp2)
            h = _call_rows(_update_head_body, N, _BN, [h, parts, na],
                           [D, (2, D), A],
                           [wu1, bu1r, wu2, bu2r,
                            wp1, bp1.reshape(1, D), wp2, bp2.reshape(1, D)],
                           D)
    return h
